# Initial kernel scaffold; baseline (speedup 1.0000x reference)
import jax
import jax.numpy as jnp
from jax.experimental import pallas as pl


def kernel(x, batch):
    raise NotImplementedError("write your pallas kernel here")


# speedup vs baseline: 3.3383x; 3.3383x over previous
import functools

import jax
import jax.numpy as jnp
from jax import lax
from jax.experimental import pallas as pl
from jax.experimental.pallas import tpu as pltpu
from jax.experimental.pallas import tpu_sc as plsc

N_ROWS = 100000
N_FEAT = 128
N_SEG = 512
NC = 2
NS = 16
COLS = N_FEAT // NC
SEG_PER_TILE = N_SEG // NS
CHUNK = 128
ROWS_MAIN = 6256
ROWS_LAST = N_ROWS - (NS - 1) * ROWS_MAIN
NFULL = ROWS_LAST // CHUNK
REM_MAIN = ROWS_MAIN - NFULL * CHUNK
REM_LAST = ROWS_LAST - NFULL * CHUNK

_mesh = plsc.VectorSubcoreMesh(core_axis_name="c", subcore_axis_name="s")


@functools.partial(
    pl.kernel,
    out_type=jax.ShapeDtypeStruct((N_SEG, N_FEAT), jnp.float32),
    mesh=_mesh,
    scratch_types=[
        pltpu.VMEM_SHARED((N_SEG, COLS), jnp.float32),
        pltpu.VMEM((CHUNK, COLS), jnp.float32),
        pltpu.VMEM((1, CHUNK), jnp.int32),
    ],
    compiler_params=pltpu.CompilerParams(use_tc_tiling_on_sc=False),
)
def _sc_segment_sum(x_hbm, b_hbm, out_hbm, acc, xbuf, idxbuf):
    c = lax.axis_index("c")
    s = lax.axis_index("s")
    col0 = c * COLS

    zvec = jnp.zeros((16,), jnp.float32)
    def _zrow(i, carry):
        for q in range(COLS // 16):
            xbuf[i, 16 * q:16 * q + 16] = zvec
        return carry
    lax.fori_loop(0, CHUNK, _zrow, 0)
    zidx = jnp.zeros((16,), jnp.int32)
    for q in range(CHUNK // 16):
        idxbuf[0, 16 * q:16 * q + 16] = zidx

    pltpu.sync_copy(xbuf.at[pl.ds(0, SEG_PER_TILE)],
                    acc.at[pl.ds(s * SEG_PER_TILE, SEG_PER_TILE)])
    plsc.subcore_barrier()

    base = s * ROWS_MAIN

    def chunk_op(start, nrows):
        pltpu.sync_copy(b_hbm.at[pl.ds(start, nrows)],
                        idxbuf.at[0, pl.ds(0, nrows)])
        pltpu.sync_copy(x_hbm.at[pl.ds(start, nrows), pl.ds(col0, COLS)],
                        xbuf.at[pl.ds(0, nrows)])
        pltpu.sync_copy(xbuf, acc.at[idxbuf.at[0]], add=True)

    @pl.when(s < NS - 1)
    def _():
        chunk_op(base + NFULL * CHUNK, REM_MAIN)

    @pl.when(s == NS - 1)
    def _():
        chunk_op(base + NFULL * CHUNK, REM_LAST)

    def _chunk(j, carry):
        chunk_op(base + j * CHUNK, CHUNK)
        return carry
    lax.fori_loop(0, NFULL, _chunk, 0)

    plsc.subcore_barrier()
    pltpu.sync_copy(acc.at[pl.ds(s * SEG_PER_TILE, SEG_PER_TILE)],
                    out_hbm.at[pl.ds(s * SEG_PER_TILE, SEG_PER_TILE),
                               pl.ds(col0, COLS)])


def kernel(x, batch):
    return _sc_segment_sum(x, batch.astype(jnp.int32))


# speedup vs baseline: 5.5251x; 1.6551x over previous
import functools

import jax
import jax.numpy as jnp
from jax import lax
from jax.experimental import pallas as pl
from jax.experimental.pallas import tpu as pltpu
from jax.experimental.pallas import tpu_sc as plsc

N_ROWS = 100000
N_FEAT = 128
N_SEG = 512
NC = 2
NS = 16
COLS = N_FEAT // NC
SEG_PER_TILE = N_SEG // NS
CHUNK = 128
ROWS_MAIN = 6256
ROWS_LAST = N_ROWS - (NS - 1) * ROWS_MAIN
NFULL = ROWS_LAST // CHUNK
REM_MAIN = ROWS_MAIN - NFULL * CHUNK
REM_LAST = ROWS_LAST - NFULL * CHUNK

_mesh = plsc.VectorSubcoreMesh(core_axis_name="c", subcore_axis_name="s")


@functools.partial(
    pl.kernel,
    out_type=jax.ShapeDtypeStruct((N_SEG, N_FEAT), jnp.float32),
    mesh=_mesh,
    scratch_types=[
        pltpu.VMEM_SHARED((N_SEG, COLS), jnp.float32),
        pltpu.VMEM((2, CHUNK, COLS), jnp.float32),
        pltpu.VMEM((2, CHUNK), jnp.int32),
        pltpu.SemaphoreType.DMA,
        pltpu.SemaphoreType.DMA,
        pltpu.SemaphoreType.DMA,
        pltpu.SemaphoreType.DMA,
    ],
    compiler_params=pltpu.CompilerParams(use_tc_tiling_on_sc=False),
)
def _sc_segment_sum(x_hbm, b_hbm, out_hbm, acc, xbuf, idxbuf,
                    semx0, semx1, semi0, semi1):
    c = lax.axis_index("c")
    s = lax.axis_index("s")
    col0 = c * COLS
    base = s * ROWS_MAIN
    semx = (semx0, semx1)
    semi = (semi0, semi1)

    zvec = jnp.zeros((16,), jnp.float32)
    def _zrow(i, carry):
        for q in range(COLS // 16):
            xbuf[0, i, 16 * q:16 * q + 16] = zvec
        return carry
    lax.fori_loop(0, CHUNK, _zrow, 0)
    zidx = jnp.zeros((16,), jnp.int32)
    for q in range(CHUNK // 16):
        idxbuf[0, 16 * q:16 * q + 16] = zidx

    pltpu.sync_copy(xbuf.at[0, pl.ds(0, SEG_PER_TILE)],
                    acc.at[pl.ds(s * SEG_PER_TILE, SEG_PER_TILE)])
    plsc.subcore_barrier()

    def load_descs(start, b):
        return (
            pltpu.make_async_copy(b_hbm.at[pl.ds(start, CHUNK)],
                                  idxbuf.at[b], semi[b]),
            pltpu.make_async_copy(
                x_hbm.at[pl.ds(start, CHUNK), pl.ds(col0, COLS)],
                xbuf.at[b], semx[b]),
        )

    def scatter(b):
        pltpu.sync_copy(xbuf.at[b], acc.at[idxbuf.at[b]], add=True)

    def rem_chunk(nrows):
        start = base + NFULL * CHUNK
        pltpu.sync_copy(b_hbm.at[pl.ds(start, nrows)],
                        idxbuf.at[0, pl.ds(0, nrows)])
        pltpu.sync_copy(x_hbm.at[pl.ds(start, nrows), pl.ds(col0, COLS)],
                        xbuf.at[0, pl.ds(0, nrows)])
        scatter(0)

    @pl.when(s < NS - 1)
    def _():
        rem_chunk(REM_MAIN)

    @pl.when(s == NS - 1)
    def _():
        rem_chunk(REM_LAST)

    for d in load_descs(base, 0):
        d.start()
    for d in load_descs(base + CHUNK, 1):
        d.start()

    def pipe(j, carry):
        j2 = 2 * j
        for b in range(2):
            jj = j2 + b
            start = base + jj * CHUNK
            for d in load_descs(start, b):
                d.wait()
            scatter(b)

            @pl.when(jj + 2 < NFULL)
            def _():
                for d in load_descs(base + (jj + 2) * CHUNK, b):
                    d.start()
        return carry
    lax.fori_loop(0, NFULL // 2, pipe, 0)

    plsc.subcore_barrier()
    pltpu.sync_copy(acc.at[pl.ds(s * SEG_PER_TILE, SEG_PER_TILE)],
                    out_hbm.at[pl.ds(s * SEG_PER_TILE, SEG_PER_TILE),
                               pl.ds(col0, COLS)])


def kernel(x, batch):
    return _sc_segment_sum(x, batch.astype(jnp.int32))


# speedup vs baseline: 5.5544x; 1.0053x over previous
import functools

import jax
import jax.numpy as jnp
from jax import lax
from jax.experimental import pallas as pl
from jax.experimental.pallas import tpu as pltpu
from jax.experimental.pallas import tpu_sc as plsc

N_ROWS = 100000
N_FEAT = 128
N_SEG = 512
NC = 2
NS = 16
COLS = N_FEAT // NC
SEG_PER_TILE = N_SEG // NS
CHUNK = 128
NBUF = 4
ROWS_MAIN = 6256
ROWS_LAST = N_ROWS - (NS - 1) * ROWS_MAIN
NFULL = ROWS_LAST // CHUNK
REM_MAIN = ROWS_MAIN - NFULL * CHUNK
REM_LAST = ROWS_LAST - NFULL * CHUNK

_mesh = plsc.VectorSubcoreMesh(core_axis_name="c", subcore_axis_name="s")


@functools.partial(
    pl.kernel,
    out_type=jax.ShapeDtypeStruct((N_SEG, N_FEAT), jnp.float32),
    mesh=_mesh,
    scratch_types=[
        pltpu.VMEM_SHARED((N_SEG, COLS), jnp.float32),
        pltpu.VMEM((NBUF, CHUNK, COLS), jnp.float32),
        pltpu.VMEM((NBUF, CHUNK), jnp.int32),
    ] + [pltpu.SemaphoreType.DMA] * (3 * NBUF),
    compiler_params=pltpu.CompilerParams(use_tc_tiling_on_sc=False),
)
def _sc_segment_sum(x_hbm, b_hbm, out_hbm, acc, xbuf, idxbuf, *sems):
    semx = sems[0:NBUF]
    semi = sems[NBUF:2 * NBUF]
    sems_ = sems[2 * NBUF:3 * NBUF]
    c = lax.axis_index("c")
    s = lax.axis_index("s")
    col0 = c * COLS
    base = s * ROWS_MAIN

    zvec = jnp.zeros((16,), jnp.float32)
    def _zrow(i, carry):
        for q in range(COLS // 16):
            xbuf[0, i, 16 * q:16 * q + 16] = zvec
        return carry
    lax.fori_loop(0, CHUNK, _zrow, 0)
    zidx = jnp.zeros((16,), jnp.int32)
    for q in range(CHUNK // 16):
        idxbuf[0, 16 * q:16 * q + 16] = zidx

    pltpu.sync_copy(xbuf.at[0, pl.ds(0, SEG_PER_TILE)],
                    acc.at[pl.ds(s * SEG_PER_TILE, SEG_PER_TILE)])
    plsc.subcore_barrier()

    def load_descs(jj, b):
        start = base + jj * CHUNK
        return (
            pltpu.make_async_copy(b_hbm.at[pl.ds(start, CHUNK)],
                                  idxbuf.at[b], semi[b]),
            pltpu.make_async_copy(
                x_hbm.at[pl.ds(start, CHUNK), pl.ds(col0, COLS)],
                xbuf.at[b], semx[b]),
        )

    def start_scatter(b):
        pltpu.async_copy(xbuf.at[b], acc.at[idxbuf.at[b]], sems_[b],
                         add=True)

    def wait_scatter(b):
        pltpu.make_async_copy(xbuf.at[b], acc.at[pl.ds(0, CHUNK)],
                              sems_[b]).wait()

    def rem_chunk(nrows):
        start = base + NFULL * CHUNK
        pltpu.sync_copy(b_hbm.at[pl.ds(start, nrows)],
                        idxbuf.at[0, pl.ds(0, nrows)])
        pltpu.sync_copy(x_hbm.at[pl.ds(start, nrows), pl.ds(col0, COLS)],
                        xbuf.at[0, pl.ds(0, nrows)])
        pltpu.sync_copy(xbuf.at[0], acc.at[idxbuf.at[0]], add=True)

    @pl.when(s < NS - 1)
    def _():
        rem_chunk(REM_MAIN)

    @pl.when(s == NS - 1)
    def _():
        rem_chunk(REM_LAST)

    for b in range(2):
        for d in load_descs(b, b):
            d.start()

    def pipe(j, carry):
        for b in range(NBUF):
            jj = NBUF * j + b
            for d in load_descs(jj, b):
                d.wait()
            start_scatter(b)
            nxt = (b + 2) % NBUF

            @pl.when((jj >= 2) & (jj + 2 < NFULL))
            def _():
                wait_scatter(nxt)
                for d in load_descs(jj + 2, nxt):
                    d.start()

            @pl.when(jj < 2)
            def _():
                for d in load_descs(jj + 2, nxt):
                    d.start()
        return carry
    lax.fori_loop(0, NFULL // NBUF, pipe, 0)
    for b in range(NBUF):
        wait_scatter((NFULL - NBUF + b) % NBUF)

    plsc.subcore_barrier()
    pltpu.sync_copy(acc.at[pl.ds(s * SEG_PER_TILE, SEG_PER_TILE)],
                    out_hbm.at[pl.ds(s * SEG_PER_TILE, SEG_PER_TILE),
                               pl.ds(col0, COLS)])


def kernel(x, batch):
    return _sc_segment_sum(x, batch.astype(jnp.int32))
